# Initial kernel scaffold; baseline (speedup 1.0000x reference)
import jax
import jax.numpy as jnp
from jax.experimental import pallas as pl


def kernel(pred_logits, pred_boxes, tgt_boxes, tgt_labels):
    raise NotImplementedError("write your pallas kernel here")


# speedup vs baseline: 10.0294x; 10.0294x over previous
import jax
import jax.numpy as jnp
from jax.experimental import pallas as pl

_B, _Q, _T = 64, 300, 50
_C = 5
_EOS = 0.1
_W_BBOX, _W_GIOU = 5.0, 2.0


def _loss_kernel(lg_ref, pb_ref, tb_ref, tl_ref, out_ref):
    lg = [lg_ref[c] for c in range(_C)]
    m = lg[0]
    for c in range(1, _C):
        m = jnp.maximum(m, lg[c])
    e = [jnp.exp(lg[c] - m) for c in range(_C)]
    s = e[0]
    for c in range(1, _C):
        s = s + e[c]
    inv_s = 1.0 / s
    prob = [e[c] * inv_s for c in range(_C)]
    log_s = jnp.log(s)
    logp = [lg[c] - m - log_s for c in range(_C)]

    pcx, pcy, pw, ph = pb_ref[0], pb_ref[1], pb_ref[2], pb_ref[3]
    px0 = pcx - 0.5 * pw
    py0 = pcy - 0.5 * ph
    px1 = pcx + 0.5 * pw
    py1 = pcy + 0.5 * ph
    parea = (px1 - px0) * (py1 - py0)

    iota_q = jax.lax.broadcasted_iota(jnp.int32, (_B, _Q), 1)

    def body(j, carry):
        used_pen, acc_ce, acc_l1, acc_gi = carry
        tbx = tb_ref[j]
        tl = tl_ref[j]
        tx0 = tbx[0][:, None]
        ty0 = tbx[1][:, None]
        tx1 = tbx[2][:, None]
        ty1 = tbx[3][:, None]
        tlc = tl[:, None]

        cls_sel = jnp.zeros((_B, _Q), jnp.float32)
        lp_sel = jnp.zeros((_B, _Q), jnp.float32)
        for c in range(_C - 1):
            mask_c = (tlc == c).astype(jnp.float32)
            cls_sel = cls_sel + mask_c * prob[c]
            lp_sel = lp_sel + mask_c * logp[c]
        cost_class = -cls_sel

        tcx = (tx0 + tx1) / 2.0
        tcy = (ty0 + ty1) / 2.0
        tw = tx1 - tx0
        th = ty1 - ty0
        cost_bbox = (jnp.abs(pcx - tcx) + jnp.abs(pcy - tcy)
                     + jnp.abs(pw - tw) + jnp.abs(ph - th))

        inter = (jnp.maximum(jnp.minimum(px1, tx1) - jnp.maximum(px0, tx0), 0.0)
                 * jnp.maximum(jnp.minimum(py1, ty1) - jnp.maximum(py0, ty0), 0.0))
        tarea = (tx1 - tx0) * (ty1 - ty0)
        union = parea + tarea - inter
        iou = inter / union
        areai = (jnp.maximum(jnp.maximum(px1, tx1) - jnp.minimum(px0, tx0), 0.0)
                 * jnp.maximum(jnp.maximum(py1, ty1) - jnp.minimum(py0, ty0), 0.0))
        giou = iou - (areai - union) / areai

        col = (_W_BBOX * cost_bbox + cost_class + _W_GIOU * (-giou)) + used_pen

        mn = jnp.min(col, axis=1, keepdims=True)
        idx = jnp.min(jnp.where(col == mn, iota_q, _Q), axis=1)
        onef = (iota_q == idx[:, None]).astype(jnp.float32)
        used_pen = used_pen + onef * 1e9

        scx = jnp.sum(pcx * onef, axis=1)
        scy = jnp.sum(pcy * onef, axis=1)
        sw = jnp.sum(pw * onef, axis=1)
        sh = jnp.sum(ph * onef, axis=1)

        tcx1, tcy1 = tcx[:, 0], tcy[:, 0]
        tw1, th1 = tw[:, 0], th[:, 0]
        l1 = (jnp.abs(scx - tcx1) + jnp.abs(scy - tcy1)
              + jnp.abs(sw - tw1) + jnp.abs(sh - th1))
        acc_l1 = acc_l1 + jnp.sum(l1)

        sx0 = scx - 0.5 * sw
        sy0 = scy - 0.5 * sh
        sx1 = scx + 0.5 * sw
        sy1 = scy + 0.5 * sh
        u0, v0, u1, v1 = tx0[:, 0], ty0[:, 0], tx1[:, 0], ty1[:, 0]
        inter_e = (jnp.maximum(jnp.minimum(sx1, u1) - jnp.maximum(sx0, u0), 0.0)
                   * jnp.maximum(jnp.minimum(sy1, v1) - jnp.maximum(sy0, v0), 0.0))
        sarea = (sx1 - sx0) * (sy1 - sy0)
        ta1 = (u1 - u0) * (v1 - v0)
        union_e = sarea + ta1 - inter_e
        iou_e = inter_e / union_e
        areai_e = (jnp.maximum(jnp.maximum(sx1, u1) - jnp.minimum(sx0, u0), 0.0)
                   * jnp.maximum(jnp.maximum(sy1, v1) - jnp.minimum(sy0, v0), 0.0))
        giou_e = iou_e - (areai_e - union_e) / areai_e
        acc_gi = acc_gi + jnp.sum(1.0 - giou_e)

        picked_l = jnp.sum(lp_sel * onef, axis=1)
        picked_n = jnp.sum(logp[_C - 1] * onef, axis=1)
        acc_ce = acc_ce + jnp.sum(_EOS * picked_n - picked_l)

        return used_pen, acc_ce, acc_l1, acc_gi

    init = (jnp.zeros((_B, _Q), jnp.float32), jnp.float32(0.0),
            jnp.float32(0.0), jnp.float32(0.0))
    _, acc_ce, acc_l1, acc_gi = jax.lax.fori_loop(0, _T, body, init)

    dense_ce = _EOS * jnp.sum(-logp[_C - 1])
    denom = jnp.float32(_B * _Q - _B * _T) * jnp.float32(_EOS) + jnp.float32(_B * _T)
    num_boxes = jnp.float32(_B * _T)
    loss_ce = (dense_ce + acc_ce) / denom
    loss_bbox = acc_l1 / num_boxes
    loss_giou = acc_gi / num_boxes
    out_ref[...] = jnp.stack([loss_ce, _W_BBOX * loss_bbox, _W_GIOU * loss_giou])


def kernel(pred_logits, pred_boxes, tgt_boxes, tgt_labels):
    lg = jnp.transpose(pred_logits, (2, 0, 1))
    pb = jnp.transpose(pred_boxes, (2, 0, 1))
    tb = jnp.transpose(tgt_boxes, (1, 2, 0))
    tl = jnp.transpose(tgt_labels, (1, 0)).astype(jnp.int32)
    return pl.pallas_call(
        _loss_kernel,
        out_shape=jax.ShapeDtypeStruct((3,), jnp.float32),
    )(lg, pb, tb, tl)
